# baseline (speedup 1.0000x reference)
import jax
import jax.numpy as jnp
from jax.experimental import pallas as pl
from jax.experimental.pallas import tpu as pltpu

_NE = 8
_K = 2
_BLK = 2048
_NBUF = 3


def _top2_softmax(logits):
    iota = jax.lax.broadcasted_iota(jnp.int32, logits.shape, 1)
    m1 = jnp.max(logits, axis=-1, keepdims=True)
    i1 = jnp.min(jnp.where(logits == m1, iota, _NE), axis=-1, keepdims=True)
    masked = jnp.where(iota == i1, jnp.float32(-jnp.inf), logits)
    m2 = jnp.max(masked, axis=-1, keepdims=True)
    i2 = jnp.min(jnp.where(masked == m2, iota, _NE), axis=-1, keepdims=True)
    t = jnp.exp(m2 - m1)
    denom = 1.0 + t
    probs = jnp.concatenate([1.0 / denom, t / denom], axis=-1)
    idx = jnp.concatenate([i1, i2], axis=-1)
    return probs, idx


def _router_kernel(hB_hbm, hA_ref, w_ref, probs_ref, idx_ref, buf, sems):
    i = pl.program_id(0)
    steps = pl.num_programs(0)
    half_blocks = steps
    w = w_ref[...]

    @pl.when(i == 0)
    def _prologue():
        for j in range(_NBUF - 1):
            pltpu.make_async_copy(
                hB_hbm.at[pl.ds((half_blocks + j) * _BLK, _BLK), :],
                buf.at[j],
                sems.at[j],
            ).start()

    @pl.when(i + _NBUF - 1 < steps)
    def _prefetch():
        nxt = i + _NBUF - 1
        pltpu.make_async_copy(
            hB_hbm.at[pl.ds((half_blocks + nxt) * _BLK, _BLK), :],
            buf.at[nxt % _NBUF],
            sems.at[nxt % _NBUF],
        ).start()

    hA = hA_ref[...]
    logitsA = jax.lax.dot_general(
        hA, w, (((1,), (1,)), ((), ())), preferred_element_type=jnp.float32
    )
    pA, iA = _top2_softmax(logitsA)
    probs_ref[0] = pA
    idx_ref[0] = iA

    slot = i % _NBUF
    pltpu.make_async_copy(
        hB_hbm.at[pl.ds((half_blocks + i) * _BLK, _BLK), :],
        buf.at[slot],
        sems.at[slot],
    ).wait()
    hB = buf[slot]
    logitsB = jax.lax.dot_general(
        hB, w, (((1,), (1,)), ((), ())), preferred_element_type=jnp.float32
    )
    pB, iB = _top2_softmax(logitsB)
    probs_ref[1] = pB
    idx_ref[1] = iB


@jax.jit
def kernel(hidden_states, weight):
    h = hidden_states.reshape(-1, hidden_states.shape[-1])
    n, hd = h.shape
    half = n // 2
    steps = half // _BLK
    probs2, idx2 = pl.pallas_call(
        _router_kernel,
        grid=(steps,),
        in_specs=[
            pl.BlockSpec(memory_space=pltpu.MemorySpace.HBM),
            pl.BlockSpec((_BLK, hd), lambda i: (i, 0)),
            pl.BlockSpec((_NE, hd), lambda i: (0, 0)),
        ],
        out_specs=[
            pl.BlockSpec((2, _BLK, _K), lambda i: (0, i, 0)),
            pl.BlockSpec((2, _BLK, _K), lambda i: (0, i, 0)),
        ],
        out_shape=[
            jax.ShapeDtypeStruct((2, half, _K), jnp.float32),
            jax.ShapeDtypeStruct((2, half, _K), jnp.int32),
        ],
        scratch_shapes=[
            pltpu.VMEM((_NBUF, _BLK, hd), jnp.float32),
            pltpu.SemaphoreType.DMA((_NBUF,)),
        ],
        compiler_params=pltpu.CompilerParams(
            dimension_semantics=("arbitrary",),
            vmem_limit_bytes=100 * 1024 * 1024,
        ),
    )(h, h, weight)
    return probs2.reshape(n, _K), idx2.reshape(n, _K)


# speedup vs baseline: 1.0269x; 1.0269x over previous
import jax
import jax.numpy as jnp
from jax.experimental import pallas as pl
from jax.experimental.pallas import tpu as pltpu

_NE = 8
_K = 2


def _top2_softmax(logits):
    iota = jax.lax.broadcasted_iota(jnp.int32, logits.shape, 1)
    m1 = jnp.max(logits, axis=-1, keepdims=True)
    i1 = jnp.min(jnp.where(logits == m1, iota, _NE), axis=-1, keepdims=True)
    masked = jnp.where(iota == i1, jnp.float32(-jnp.inf), logits)
    m2 = jnp.max(masked, axis=-1, keepdims=True)
    i2 = jnp.min(jnp.where(masked == m2, iota, _NE), axis=-1, keepdims=True)
    t = jnp.exp(m2 - m1)
    denom = 1.0 + t
    probs = jnp.concatenate([1.0 / denom, t / denom], axis=-1)
    idx = jnp.concatenate([i1, i2], axis=-1)
    return probs, idx


def _router_kernel(h_ref, w_ref, probs_ref, idx_ref):
    h = h_ref[...]
    w = w_ref[...]
    logits = jax.lax.dot_general(
        h, w, (((1,), (1,)), ((), ())), preferred_element_type=jnp.float32
    )
    probs, idx = _top2_softmax(logits)
    probs_ref[...] = probs
    idx_ref[...] = idx


@jax.jit
def kernel(hidden_states, weight):
    h = hidden_states.reshape(-1, hidden_states.shape[-1])
    n, hd = h.shape
    blk = 2048
    probs, idx = pl.pallas_call(
        _router_kernel,
        grid=(n // blk,),
        in_specs=[
            pl.BlockSpec((blk, hd), lambda i: (i, 0)),
            pl.BlockSpec((_NE, hd), lambda i: (0, 0)),
        ],
        out_specs=[
            pl.BlockSpec((blk, _K), lambda i: (i, 0)),
            pl.BlockSpec((blk, _K), lambda i: (i, 0)),
        ],
        out_shape=[
            jax.ShapeDtypeStruct((n, _K), jnp.float32),
            jax.ShapeDtypeStruct((n, _K), jnp.int32),
        ],
        compiler_params=pltpu.CompilerParams(
            dimension_semantics=("arbitrary",),
            vmem_limit_bytes=100 * 1024 * 1024,
        ),
    )(h, weight)
    return probs, idx
